# baseline (speedup 1.0000x reference)
import functools

import jax
import jax.numpy as jnp
from jax import lax
from jax.experimental import pallas as pl
from jax.experimental.pallas import tpu as pltpu
from jax.experimental.pallas import tpu_sc as plsc

V = 100000
D = 128
B = 4096
L = 200

NC = 2
NS = 16
NW = NC * NS
ROWS_PER_W = B // NW
GROUPS = ROWS_PER_W // 16


_NSTREAM = 4
_VSUB = 512
_VSTEP = _NSTREAM * _VSUB
_VGRID = -(-V // _VSTEP)
_VPAD = _VGRID * _VSTEP


def _project_body(*refs):
    tab_refs = refs[:_NSTREAM]
    w_ref, b_ref, o_ref, b16_ref = refs[_NSTREAM:]
    i = pl.program_id(0)
    w = w_ref[...]
    for k in range(_NSTREAM):
        s = jnp.sum(tab_refs[k][...] * w, axis=1)
        o_ref[pl.ds(i * _VSTEP + k * _VSUB, _VSUB)] = s
    b16_ref[...] = jnp.broadcast_to(b_ref[...], (16,))


def _project_table(table, W, b):
    def _tab_spec(k):
        return pl.BlockSpec((_VSUB, D), lambda i, k=k: (_NSTREAM * i + k, 0))

    return pl.pallas_call(
        _project_body,
        grid=(_VGRID,),
        in_specs=[_tab_spec(k) for k in range(_NSTREAM)] + [
            pl.BlockSpec((1, D), lambda i: (0, 0)),
            pl.BlockSpec((1,), lambda i: (0,)),
        ],
        out_specs=[
            pl.BlockSpec((_VPAD,), lambda i: (0,)),
            pl.BlockSpec((16,), lambda i: (0,)),
        ],
        out_shape=[
            jax.ShapeDtypeStruct((_VPAD,), jnp.float32),
            jax.ShapeDtypeStruct((16,), jnp.float32),
        ],
    )(*([table] * _NSTREAM), W, b)


def _pool_body(t_hbm, x_hbm, b_hbm, out_hbm, t_v, x_v, b_v, out_v,
               sem_t, sem_x):
    wid = lax.axis_index("s") * NC + lax.axis_index("c")
    base = wid * ROWS_PER_W

    cp_t = pltpu.make_async_copy(t_hbm, t_v, sem_t)
    cp_t.start()
    cp_x = pltpu.make_async_copy(
        x_hbm.at[pl.ds(base * L, ROWS_PER_W * L)], x_v, sem_x)
    cp_x.start()
    pltpu.sync_copy(b_hbm, b_v)
    cp_x.wait()
    cp_t.wait()

    lane = lax.broadcasted_iota(jnp.int32, (16,), 0)
    zero = jnp.zeros((16,), jnp.float32)
    izero = jnp.zeros((16,), jnp.int32)
    one = jnp.ones((16,), jnp.int32)
    b_vec = b_v[...]

    for g in range(GROUPS):
        flat0 = (g * 16 + lane) * L

        def body(j, carry):
            acc, cnt = carry
            xi = plsc.load_gather(x_v, [flat0 + j])
            tv = plsc.load_gather(t_v, [xi])
            acc = acc + tv
            cnt = cnt + jnp.minimum(xi, one)
            return acc, cnt

        acc, cnt = lax.fori_loop(0, L, body, (zero, izero), unroll=25)
        cntf = jnp.maximum(cnt.astype(jnp.float32), 1.0)
        out_v[pl.ds(g * 16, 16)] = acc / cntf + b_vec

    pltpu.sync_copy(out_v, out_hbm.at[pl.ds(base, ROWS_PER_W)])


@functools.cache
def _make_pool_kernel():
    mesh = plsc.VectorSubcoreMesh(core_axis_name="c", subcore_axis_name="s")
    return pl.kernel(
        _pool_body,
        out_type=jax.ShapeDtypeStruct((B,), jnp.float32),
        mesh=mesh,
        scratch_types=[
            pltpu.VMEM((_VPAD,), jnp.float32),
            pltpu.VMEM((ROWS_PER_W * L,), jnp.int32),
            pltpu.VMEM((16,), jnp.float32),
            pltpu.VMEM((ROWS_PER_W,), jnp.float32),
            pltpu.SemaphoreType.DMA,
            pltpu.SemaphoreType.DMA,
        ],
        compiler_params=pltpu.CompilerParams(needs_layout_passes=False),
    )


def kernel(x, table, W, b):
    t, b16 = _project_table(table, W, b)
    return _make_pool_kernel()(t, x.reshape(B * L), b16)


# speedup vs baseline: 1.0233x; 1.0233x over previous
import functools

import jax
import jax.numpy as jnp
from jax import lax
from jax.experimental import pallas as pl
from jax.experimental.pallas import tpu as pltpu
from jax.experimental.pallas import tpu_sc as plsc

V = 100000
D = 128
B = 4096
L = 200

NC = 2
NS = 16
NW = NC * NS
ROWS_PER_W = B // NW
GROUPS = ROWS_PER_W // 16


_NSTREAM = 1
_VSUB = 2048
_VSTEP = _NSTREAM * _VSUB
_VGRID = -(-V // _VSTEP)
_VPAD = _VGRID * _VSTEP


def _project_body(*refs):
    tab_refs = refs[:_NSTREAM]
    w_ref, b_ref, o_ref, b16_ref = refs[_NSTREAM:]
    i = pl.program_id(0)
    w = w_ref[...]
    for k in range(_NSTREAM):
        s = jnp.sum(tab_refs[k][...] * w, axis=1)
        o_ref[pl.ds(i * _VSTEP + k * _VSUB, _VSUB)] = s
    b16_ref[...] = jnp.broadcast_to(b_ref[...], (16,))


def _project_table(table, W, b):
    def _tab_spec(k):
        return pl.BlockSpec((_VSUB, D), lambda i, k=k: (_NSTREAM * i + k, 0))

    return pl.pallas_call(
        _project_body,
        grid=(_VGRID,),
        in_specs=[_tab_spec(k) for k in range(_NSTREAM)] + [
            pl.BlockSpec((1, D), lambda i: (0, 0)),
            pl.BlockSpec((1,), lambda i: (0,)),
        ],
        out_specs=[
            pl.BlockSpec((_VPAD,), lambda i: (0,)),
            pl.BlockSpec((16,), lambda i: (0,)),
        ],
        out_shape=[
            jax.ShapeDtypeStruct((_VPAD,), jnp.float32),
            jax.ShapeDtypeStruct((16,), jnp.float32),
        ],
    )(*([table] * _NSTREAM), W, b)


def _pool_body(t_hbm, x_hbm, b_hbm, out_hbm, t_v, x_v, b_v, out_v,
               sem_t, sem_x):
    wid = lax.axis_index("s") * NC + lax.axis_index("c")
    base = wid * ROWS_PER_W

    cp_t = pltpu.make_async_copy(t_hbm, t_v, sem_t)
    cp_t.start()
    cp_x = pltpu.make_async_copy(
        x_hbm.at[pl.ds(base * L, ROWS_PER_W * L)], x_v, sem_x)
    cp_x.start()
    pltpu.sync_copy(b_hbm, b_v)
    cp_x.wait()
    cp_t.wait()

    lane = lax.broadcasted_iota(jnp.int32, (16,), 0)
    zero = jnp.zeros((16,), jnp.float32)
    izero = jnp.zeros((16,), jnp.int32)
    one = jnp.ones((16,), jnp.int32)
    b_vec = b_v[...]

    def group_body(g, _):
        flat0 = (g * 16 + lane) * L

        def body(j, carry):
            acc, cnt = carry
            xi = plsc.load_gather(x_v, [flat0 + j])
            tv = plsc.load_gather(t_v, [xi])
            acc = acc + tv
            cnt = cnt + jnp.minimum(xi, one)
            return acc, cnt

        acc, cnt = lax.fori_loop(0, L, body, (zero, izero), unroll=25)
        cntf = jnp.maximum(cnt.astype(jnp.float32), 1.0)
        out_v[pl.ds(pl.multiple_of(g * 16, 16), 16)] = acc / cntf + b_vec
        return 0

    lax.fori_loop(0, GROUPS, group_body, 0)

    pltpu.sync_copy(out_v, out_hbm.at[pl.ds(base, ROWS_PER_W)])


@functools.cache
def _make_pool_kernel():
    mesh = plsc.VectorSubcoreMesh(core_axis_name="c", subcore_axis_name="s")
    return pl.kernel(
        _pool_body,
        out_type=jax.ShapeDtypeStruct((B,), jnp.float32),
        mesh=mesh,
        scratch_types=[
            pltpu.VMEM((_VPAD,), jnp.float32),
            pltpu.VMEM((ROWS_PER_W * L,), jnp.int32),
            pltpu.VMEM((16,), jnp.float32),
            pltpu.VMEM((ROWS_PER_W,), jnp.float32),
            pltpu.SemaphoreType.DMA,
            pltpu.SemaphoreType.DMA,
        ],
        compiler_params=pltpu.CompilerParams(needs_layout_passes=False),
    )


def kernel(x, table, W, b):
    t, b16 = _project_table(table, W, b)
    return _make_pool_kernel()(t, x.reshape(B * L), b16)


# speedup vs baseline: 1.1152x; 1.0898x over previous
import functools

import jax
import jax.numpy as jnp
from jax import lax
from jax.experimental import pallas as pl
from jax.experimental.pallas import tpu as pltpu
from jax.experimental.pallas import tpu_sc as plsc

V = 100000
D = 128
B = 4096
L = 200

NC = 2
NS = 16
NW = NC * NS
ROWS_PER_W = B // NW
GROUPS = ROWS_PER_W // 16


_NSTREAM = 1
_VSUB = 4096
_VSTEP = _NSTREAM * _VSUB
_VGRID = -(-V // _VSTEP)
_VPAD = _VGRID * _VSTEP


def _project_body(*refs):
    tab_refs = refs[:_NSTREAM]
    w_ref, b_ref, o_ref, b16_ref = refs[_NSTREAM:]
    i = pl.program_id(0)
    w = w_ref[...]
    for k in range(_NSTREAM):
        s = jnp.sum(tab_refs[k][...] * w, axis=1)
        o_ref[pl.ds(i * _VSTEP + k * _VSUB, _VSUB)] = s
    b16_ref[...] = jnp.broadcast_to(b_ref[...], (16,))


def _project_table(table, W, b):
    def _tab_spec(k):
        return pl.BlockSpec((_VSUB, D), lambda i, k=k: (_NSTREAM * i + k, 0))

    return pl.pallas_call(
        _project_body,
        grid=(_VGRID,),
        in_specs=[_tab_spec(k) for k in range(_NSTREAM)] + [
            pl.BlockSpec((1, D), lambda i: (0, 0)),
            pl.BlockSpec((1,), lambda i: (0,)),
        ],
        out_specs=[
            pl.BlockSpec((_VPAD,), lambda i: (0,)),
            pl.BlockSpec((16,), lambda i: (0,)),
        ],
        out_shape=[
            jax.ShapeDtypeStruct((_VPAD,), jnp.float32),
            jax.ShapeDtypeStruct((16,), jnp.float32),
        ],
    )(*([table] * _NSTREAM), W, b)


def _pool_body(t_hbm, x_hbm, b_hbm, out_hbm, t_v, x_v, b_v, out_v,
               sem_t, sem_x):
    wid = lax.axis_index("s") * NC + lax.axis_index("c")
    base = wid * ROWS_PER_W

    cp_t = pltpu.make_async_copy(t_hbm, t_v, sem_t)
    cp_t.start()
    cp_x = pltpu.make_async_copy(
        x_hbm.at[pl.ds(base * L, ROWS_PER_W * L)], x_v, sem_x)
    cp_x.start()
    pltpu.sync_copy(b_hbm, b_v)
    cp_x.wait()
    cp_t.wait()

    lane = lax.broadcasted_iota(jnp.int32, (16,), 0)
    zero = jnp.zeros((16,), jnp.float32)
    izero = jnp.zeros((16,), jnp.int32)
    one = jnp.ones((16,), jnp.int32)
    b_vec = b_v[...]

    def group_body(g, _):
        flat0 = (g * 16 + lane) * L

        def body(j, carry):
            acc, cnt = carry
            xi = plsc.load_gather(x_v, [flat0 + j])
            tv = plsc.load_gather(t_v, [xi])
            acc = acc + tv
            cnt = cnt + jnp.minimum(xi, one)
            return acc, cnt

        acc, cnt = lax.fori_loop(0, L, body, (zero, izero), unroll=25)
        cntf = jnp.maximum(cnt.astype(jnp.float32), 1.0)
        out_v[pl.ds(pl.multiple_of(g * 16, 16), 16)] = acc / cntf + b_vec
        return 0

    lax.fori_loop(0, GROUPS, group_body, 0)

    pltpu.sync_copy(out_v, out_hbm.at[pl.ds(base, ROWS_PER_W)])


@functools.cache
def _make_pool_kernel():
    mesh = plsc.VectorSubcoreMesh(core_axis_name="c", subcore_axis_name="s")
    return pl.kernel(
        _pool_body,
        out_type=jax.ShapeDtypeStruct((B,), jnp.float32),
        mesh=mesh,
        scratch_types=[
            pltpu.VMEM((_VPAD,), jnp.float32),
            pltpu.VMEM((ROWS_PER_W * L,), jnp.int32),
            pltpu.VMEM((16,), jnp.float32),
            pltpu.VMEM((ROWS_PER_W,), jnp.float32),
            pltpu.SemaphoreType.DMA,
            pltpu.SemaphoreType.DMA,
        ],
        compiler_params=pltpu.CompilerParams(needs_layout_passes=False),
    )


def kernel(x, table, W, b):
    t, b16 = _project_table(table, W, b)
    return _make_pool_kernel()(t, x.reshape(B * L), b16)


# speedup vs baseline: 1.1367x; 1.0193x over previous
import functools

import jax
import jax.numpy as jnp
from jax import lax
from jax.experimental import pallas as pl
from jax.experimental.pallas import tpu as pltpu
from jax.experimental.pallas import tpu_sc as plsc

V = 100000
D = 128
B = 4096
L = 200

NC = 2
NS = 16
NW = NC * NS
ROWS_PER_W = B // NW
GROUPS = ROWS_PER_W // 16


_NSTREAM = 1
_VSUB = 6400
_VSTEP = _NSTREAM * _VSUB
_VGRID = -(-V // _VSTEP)
_VPAD = _VGRID * _VSTEP


def _project_body(*refs):
    tab_refs = refs[:_NSTREAM]
    w_ref, b_ref, o_ref, b16_ref = refs[_NSTREAM:]
    i = pl.program_id(0)
    w = w_ref[...]
    for k in range(_NSTREAM):
        s = jnp.sum(tab_refs[k][...] * w, axis=1)
        o_ref[pl.ds(i * _VSTEP + k * _VSUB, _VSUB)] = s
    b16_ref[...] = jnp.broadcast_to(b_ref[...], (16,))


def _project_table(table, W, b):
    def _tab_spec(k):
        return pl.BlockSpec((_VSUB, D), lambda i, k=k: (_NSTREAM * i + k, 0))

    return pl.pallas_call(
        _project_body,
        grid=(_VGRID,),
        in_specs=[_tab_spec(k) for k in range(_NSTREAM)] + [
            pl.BlockSpec((1, D), lambda i: (0, 0)),
            pl.BlockSpec((1,), lambda i: (0,)),
        ],
        out_specs=[
            pl.BlockSpec((_VPAD,), lambda i: (0,)),
            pl.BlockSpec((16,), lambda i: (0,)),
        ],
        out_shape=[
            jax.ShapeDtypeStruct((_VPAD,), jnp.float32),
            jax.ShapeDtypeStruct((16,), jnp.float32),
        ],
    )(*([table] * _NSTREAM), W, b)


def _pool_body(t_hbm, x_hbm, b_hbm, out_hbm, t_v, x_v, b_v, out_v,
               sem_t, sem_x):
    wid = lax.axis_index("s") * NC + lax.axis_index("c")
    base = wid * ROWS_PER_W

    cp_t = pltpu.make_async_copy(t_hbm, t_v, sem_t)
    cp_t.start()
    cp_x = pltpu.make_async_copy(
        x_hbm.at[pl.ds(base * L, ROWS_PER_W * L)], x_v, sem_x)
    cp_x.start()
    pltpu.sync_copy(b_hbm, b_v)
    cp_x.wait()
    cp_t.wait()

    lane = lax.broadcasted_iota(jnp.int32, (16,), 0)
    zero = jnp.zeros((16,), jnp.float32)
    izero = jnp.zeros((16,), jnp.int32)
    one = jnp.ones((16,), jnp.int32)
    b_vec = b_v[...]

    def group_body(g, _):
        flat0 = (g * 16 + lane) * L

        def body(j, carry):
            acc, cnt = carry
            xi = plsc.load_gather(x_v, [flat0 + j])
            tv = plsc.load_gather(t_v, [xi])
            acc = acc + tv
            cnt = cnt + jnp.minimum(xi, one)
            return acc, cnt

        acc, cnt = lax.fori_loop(0, L, body, (zero, izero), unroll=25)
        cntf = jnp.maximum(cnt.astype(jnp.float32), 1.0)
        out_v[pl.ds(pl.multiple_of(g * 16, 16), 16)] = acc / cntf + b_vec
        return 0

    lax.fori_loop(0, GROUPS, group_body, 0)

    pltpu.sync_copy(out_v, out_hbm.at[pl.ds(base, ROWS_PER_W)])


@functools.cache
def _make_pool_kernel():
    mesh = plsc.VectorSubcoreMesh(core_axis_name="c", subcore_axis_name="s")
    return pl.kernel(
        _pool_body,
        out_type=jax.ShapeDtypeStruct((B,), jnp.float32),
        mesh=mesh,
        scratch_types=[
            pltpu.VMEM((_VPAD,), jnp.float32),
            pltpu.VMEM((ROWS_PER_W * L,), jnp.int32),
            pltpu.VMEM((16,), jnp.float32),
            pltpu.VMEM((ROWS_PER_W,), jnp.float32),
            pltpu.SemaphoreType.DMA,
            pltpu.SemaphoreType.DMA,
        ],
        compiler_params=pltpu.CompilerParams(needs_layout_passes=False),
    )


def kernel(x, table, W, b):
    t, b16 = _project_table(table, W, b)
    return _make_pool_kernel()(t, x.reshape(B * L), b16)


# speedup vs baseline: 1.1430x; 1.0055x over previous
import functools

import jax
import jax.numpy as jnp
from jax import lax
from jax.experimental import pallas as pl
from jax.experimental.pallas import tpu as pltpu
from jax.experimental.pallas import tpu_sc as plsc

V = 100000
D = 128
B = 4096
L = 200

NC = 2
NS = 16
NW = NC * NS
ROWS_PER_W = B // NW
GROUPS = ROWS_PER_W // 16


_NSTREAM = 1
_VSUB = 12800
_VSTEP = _NSTREAM * _VSUB
_VGRID = -(-V // _VSTEP)
_VPAD = _VGRID * _VSTEP


def _project_body(*refs):
    tab_refs = refs[:_NSTREAM]
    w_ref, b_ref, o_ref, b16_ref = refs[_NSTREAM:]
    i = pl.program_id(0)
    w = w_ref[...]
    for k in range(_NSTREAM):
        s = jnp.sum(tab_refs[k][...] * w, axis=1)
        o_ref[pl.ds(i * _VSTEP + k * _VSUB, _VSUB)] = s
    b16_ref[...] = jnp.broadcast_to(b_ref[...], (16,))


def _project_table(table, W, b):
    def _tab_spec(k):
        return pl.BlockSpec((_VSUB, D), lambda i, k=k: (_NSTREAM * i + k, 0))

    return pl.pallas_call(
        _project_body,
        grid=(_VGRID,),
        in_specs=[_tab_spec(k) for k in range(_NSTREAM)] + [
            pl.BlockSpec((1, D), lambda i: (0, 0)),
            pl.BlockSpec((1,), lambda i: (0,)),
        ],
        out_specs=[
            pl.BlockSpec((_VPAD,), lambda i: (0,)),
            pl.BlockSpec((16,), lambda i: (0,)),
        ],
        out_shape=[
            jax.ShapeDtypeStruct((_VPAD,), jnp.float32),
            jax.ShapeDtypeStruct((16,), jnp.float32),
        ],
    )(*([table] * _NSTREAM), W, b)


def _pool_body(t_hbm, x_hbm, b_hbm, out_hbm, t_v, x_v, b_v, out_v,
               sem_t, sem_x):
    wid = lax.axis_index("s") * NC + lax.axis_index("c")
    base = wid * ROWS_PER_W

    cp_t = pltpu.make_async_copy(t_hbm, t_v, sem_t)
    cp_t.start()
    cp_x = pltpu.make_async_copy(
        x_hbm.at[pl.ds(base * L, ROWS_PER_W * L)], x_v, sem_x)
    cp_x.start()
    pltpu.sync_copy(b_hbm, b_v)
    cp_x.wait()
    cp_t.wait()

    lane = lax.broadcasted_iota(jnp.int32, (16,), 0)
    zero = jnp.zeros((16,), jnp.float32)
    izero = jnp.zeros((16,), jnp.int32)
    one = jnp.ones((16,), jnp.int32)
    b_vec = b_v[...]

    def group_body(g, _):
        flat0 = (g * 16 + lane) * L

        def body(j, carry):
            acc, cnt = carry
            xi = plsc.load_gather(x_v, [flat0 + j])
            tv = plsc.load_gather(t_v, [xi])
            acc = acc + tv
            cnt = cnt + jnp.minimum(xi, one)
            return acc, cnt

        acc, cnt = lax.fori_loop(0, L, body, (zero, izero), unroll=25)
        cntf = jnp.maximum(cnt.astype(jnp.float32), 1.0)
        out_v[pl.ds(pl.multiple_of(g * 16, 16), 16)] = acc / cntf + b_vec
        return 0

    lax.fori_loop(0, GROUPS, group_body, 0)

    pltpu.sync_copy(out_v, out_hbm.at[pl.ds(base, ROWS_PER_W)])


@functools.cache
def _make_pool_kernel():
    mesh = plsc.VectorSubcoreMesh(core_axis_name="c", subcore_axis_name="s")
    return pl.kernel(
        _pool_body,
        out_type=jax.ShapeDtypeStruct((B,), jnp.float32),
        mesh=mesh,
        scratch_types=[
            pltpu.VMEM((_VPAD,), jnp.float32),
            pltpu.VMEM((ROWS_PER_W * L,), jnp.int32),
            pltpu.VMEM((16,), jnp.float32),
            pltpu.VMEM((ROWS_PER_W,), jnp.float32),
            pltpu.SemaphoreType.DMA,
            pltpu.SemaphoreType.DMA,
        ],
        compiler_params=pltpu.CompilerParams(needs_layout_passes=False),
    )


def kernel(x, table, W, b):
    t, b16 = _project_table(table, W, b)
    return _make_pool_kernel()(t, x.reshape(B * L), b16)
